# initial kernel scaffold (unmeasured)
import jax
import jax.numpy as jnp
from jax import lax
from jax.experimental import pallas as pl
from jax.experimental.pallas import tpu as pltpu

M = 4096
D = 4096
HALF = M // 2
N_CHUNKS = 8
CHUNK = HALF // N_CHUNKS


def kernel(partial, resid, gamma):
    def body(partial_ref, resid_ref, gamma_ref, out_ref, recv_hbm,
             p_vmem, q_vmem, r_vmem, o_vmem,
             local_sems, send_x_sems, recv_x_sems, send_y_sems, recv_y_sems):
        my_x = lax.axis_index("x")
        my_y = lax.axis_index("y")
        x_nbr = (1 - my_x, my_y)
        y_nbr = (my_x, 1 - my_y)

        barrier_sem = pltpu.get_barrier_semaphore()
        for nbr in (x_nbr, y_nbr):
            pl.semaphore_signal(
                barrier_sem, inc=1,
                device_id=nbr, device_id_type=pl.DeviceIdType.MESH,
            )
        pl.semaphore_wait(barrier_sem, 2)

        row0 = my_y * HALF
        other_row0 = (1 - my_y) * HALF

        x_rdmas = []
        for c in range(N_CHUNKS):
            rd = pltpu.make_async_remote_copy(
                src_ref=partial_ref.at[0, pl.ds(row0 + c * CHUNK, CHUNK), :],
                dst_ref=recv_hbm.at[pl.ds(c * CHUNK, CHUNK), :],
                send_sem=send_x_sems.at[c],
                recv_sem=recv_x_sems.at[c],
                device_id=x_nbr,
                device_id_type=pl.DeviceIdType.MESH,
            )
            rd.start()
            x_rdmas.append(rd)

        for c in range(N_CHUNKS):
            x_rdmas[c].wait_recv()
            cp_p = pltpu.make_async_copy(
                partial_ref.at[0, pl.ds(row0 + c * CHUNK, CHUNK), :],
                p_vmem, local_sems.at[0])
            cp_q = pltpu.make_async_copy(
                recv_hbm.at[pl.ds(c * CHUNK, CHUNK), :],
                q_vmem, local_sems.at[1])
            cp_r = pltpu.make_async_copy(
                resid_ref.at[pl.ds(row0 + c * CHUNK, CHUNK), :],
                r_vmem, local_sems.at[2])
            cp_p.start()
            cp_q.start()
            cp_r.start()
            cp_p.wait()
            cp_q.wait()
            cp_r.wait()

            y = p_vmem[...] + q_vmem[...] + r_vmem[...]
            rms = jnp.sqrt(jnp.mean(y * y, axis=-1, keepdims=True) + 1e-6)
            o_vmem[...] = y / rms * gamma_ref[...]

            cp_o = pltpu.make_async_copy(
                o_vmem,
                out_ref.at[pl.ds(row0 + c * CHUNK, CHUNK), :],
                local_sems.at[3])
            cp_o.start()
            rd = pltpu.make_async_remote_copy(
                src_ref=o_vmem,
                dst_ref=out_ref.at[pl.ds(row0 + c * CHUNK, CHUNK), :],
                send_sem=send_y_sems.at[c],
                recv_sem=recv_y_sems.at[c],
                device_id=y_nbr,
                device_id_type=pl.DeviceIdType.MESH,
            )
            rd.start()
            rd.wait_send()
            cp_o.wait()

        for c in range(N_CHUNKS):
            x_rdmas[c].wait_send()

        for c in range(N_CHUNKS):
            rd = pltpu.make_async_remote_copy(
                src_ref=o_vmem,
                dst_ref=out_ref.at[pl.ds(other_row0 + c * CHUNK, CHUNK), :],
                send_sem=send_y_sems.at[c],
                recv_sem=recv_y_sems.at[c],
                device_id=y_nbr,
                device_id_type=pl.DeviceIdType.MESH,
            )
            rd.wait_recv()

    gamma2 = gamma.reshape(1, D)
    return pl.pallas_call(
        body,
        out_shape=jax.ShapeDtypeStruct((M, D), jnp.float32),
        in_specs=[
            pl.BlockSpec(memory_space=pltpu.HBM),
            pl.BlockSpec(memory_space=pltpu.HBM),
            pl.BlockSpec(memory_space=pltpu.VMEM),
        ],
        out_specs=pl.BlockSpec(memory_space=pltpu.HBM),
        scratch_shapes=[
            pltpu.MemorySpace.HBM((HALF, D), jnp.float32),
            pltpu.VMEM((CHUNK, D), jnp.float32),
            pltpu.VMEM((CHUNK, D), jnp.float32),
            pltpu.VMEM((CHUNK, D), jnp.float32),
            pltpu.VMEM((CHUNK, D), jnp.float32),
            pltpu.SemaphoreType.DMA((4,)),
            pltpu.SemaphoreType.DMA((N_CHUNKS,)),
            pltpu.SemaphoreType.DMA((N_CHUNKS,)),
            pltpu.SemaphoreType.DMA((N_CHUNKS,)),
            pltpu.SemaphoreType.DMA((N_CHUNKS,)),
        ],
        compiler_params=pltpu.CompilerParams(collective_id=0),
    )(partial, resid, gamma2)


# baseline (device time: 508254 ns/iter reference)
import jax
import jax.numpy as jnp
from jax import lax
from jax.experimental import pallas as pl
from jax.experimental.pallas import tpu as pltpu

M = 4096
D = 4096
HALF = M // 2
N_CHUNKS = 8
CHUNK = HALF // N_CHUNKS


def kernel(partial, resid, gamma):
    def body(partial_ref, resid_ref, gamma_ref, out_ref, recv_hbm,
             p_vmem, q_vmem, r_vmem, o_vmem,
             local_sems, send_x_sems, recv_x_sems, send_y_sems, recv_y_sems):
        my_x = lax.axis_index("x")
        my_y = lax.axis_index("y")
        x_nbr = (1 - my_x, my_y)
        y_nbr = (my_x, 1 - my_y)

        barrier_sem = pltpu.get_barrier_semaphore()
        for nbr in (x_nbr, y_nbr):
            pl.semaphore_signal(
                barrier_sem, inc=1,
                device_id=nbr, device_id_type=pl.DeviceIdType.MESH,
            )
        pl.semaphore_wait(barrier_sem, 2)

        row0 = my_y * HALF
        other_row0 = (1 - my_y) * HALF

        x_rdmas = []
        for c in range(N_CHUNKS):
            rd = pltpu.make_async_remote_copy(
                src_ref=partial_ref.at[0, pl.ds(row0 + c * CHUNK, CHUNK), :],
                dst_ref=recv_hbm.at[pl.ds(c * CHUNK, CHUNK), :],
                send_sem=send_x_sems.at[c],
                recv_sem=recv_x_sems.at[c],
                device_id=x_nbr,
                device_id_type=pl.DeviceIdType.MESH,
            )
            rd.start()
            x_rdmas.append(rd)

        for c in range(N_CHUNKS):
            x_rdmas[c].wait_recv()
            cp_p = pltpu.make_async_copy(
                partial_ref.at[0, pl.ds(row0 + c * CHUNK, CHUNK), :],
                p_vmem, local_sems.at[0])
            cp_q = pltpu.make_async_copy(
                recv_hbm.at[pl.ds(c * CHUNK, CHUNK), :],
                q_vmem, local_sems.at[1])
            cp_r = pltpu.make_async_copy(
                resid_ref.at[pl.ds(row0 + c * CHUNK, CHUNK), :],
                r_vmem, local_sems.at[2])
            cp_p.start()
            cp_q.start()
            cp_r.start()
            cp_p.wait()
            cp_q.wait()
            cp_r.wait()

            y = p_vmem[...] + q_vmem[...] + r_vmem[...]
            rms = jnp.sqrt(jnp.mean(y * y, axis=-1, keepdims=True) + 1e-6)
            o_vmem[...] = y / rms * gamma_ref[...]

            cp_o = pltpu.make_async_copy(
                o_vmem,
                out_ref.at[pl.ds(row0 + c * CHUNK, CHUNK), :],
                local_sems.at[3])
            cp_o.start()
            rd = pltpu.make_async_remote_copy(
                src_ref=o_vmem,
                dst_ref=out_ref.at[pl.ds(row0 + c * CHUNK, CHUNK), :],
                send_sem=send_y_sems.at[c],
                recv_sem=recv_y_sems.at[c],
                device_id=y_nbr,
                device_id_type=pl.DeviceIdType.MESH,
            )
            rd.start()
            rd.wait_send()
            cp_o.wait()

        for c in range(N_CHUNKS):
            x_rdmas[c].wait_send()

        for c in range(N_CHUNKS):
            rd = pltpu.make_async_remote_copy(
                src_ref=o_vmem,
                dst_ref=out_ref.at[pl.ds(other_row0 + c * CHUNK, CHUNK), :],
                send_sem=send_y_sems.at[c],
                recv_sem=recv_y_sems.at[c],
                device_id=y_nbr,
                device_id_type=pl.DeviceIdType.MESH,
            )
            rd.wait_recv()

    gamma2 = gamma.reshape(1, D)
    out, _ = pl.pallas_call(
        body,
        out_shape=[
            jax.ShapeDtypeStruct((M, D), jnp.float32),
            jax.ShapeDtypeStruct((HALF, D), jnp.float32),
        ],
        in_specs=[
            pl.BlockSpec(memory_space=pltpu.HBM),
            pl.BlockSpec(memory_space=pltpu.HBM),
            pl.BlockSpec(memory_space=pltpu.VMEM),
        ],
        out_specs=[
            pl.BlockSpec(memory_space=pltpu.HBM),
            pl.BlockSpec(memory_space=pltpu.HBM),
        ],
        scratch_shapes=[
            pltpu.VMEM((CHUNK, D), jnp.float32),
            pltpu.VMEM((CHUNK, D), jnp.float32),
            pltpu.VMEM((CHUNK, D), jnp.float32),
            pltpu.VMEM((CHUNK, D), jnp.float32),
            pltpu.SemaphoreType.DMA((4,)),
            pltpu.SemaphoreType.DMA((N_CHUNKS,)),
            pltpu.SemaphoreType.DMA((N_CHUNKS,)),
            pltpu.SemaphoreType.DMA((N_CHUNKS,)),
            pltpu.SemaphoreType.DMA((N_CHUNKS,)),
        ],
        compiler_params=pltpu.CompilerParams(collective_id=0),
    )(partial, resid, gamma2)
    return out


# device time: 456927 ns/iter; 1.1123x vs baseline; 1.1123x over previous
import jax
import jax.numpy as jnp
from jax import lax
from jax.experimental import pallas as pl
from jax.experimental.pallas import tpu as pltpu

M = 4096
D = 4096
HALF = M // 2
N_CHUNKS = 16
CHUNK = HALF // N_CHUNKS


def kernel(partial, resid, gamma):
    def body(partial_ref, resid_ref, gamma_ref, out_ref, recv_hbm,
             p_vmem, q_vmem, r_vmem, o_vmem,
             in_sems, out_sems,
             send_x_sems, recv_x_sems, send_y_sems, recv_y_sems):
        my_x = lax.axis_index("x")
        my_y = lax.axis_index("y")
        x_nbr = (1 - my_x, my_y)
        y_nbr = (my_x, 1 - my_y)

        barrier_sem = pltpu.get_barrier_semaphore()
        for nbr in (x_nbr, y_nbr):
            pl.semaphore_signal(
                barrier_sem, inc=1,
                device_id=nbr, device_id_type=pl.DeviceIdType.MESH,
            )
        pl.semaphore_wait(barrier_sem, 2)

        row0 = my_y * HALF
        other_row0 = (1 - my_y) * HALF

        x_rdmas = []
        for c in range(N_CHUNKS):
            rd = pltpu.make_async_remote_copy(
                src_ref=partial_ref.at[0, pl.ds(row0 + c * CHUNK, CHUNK), :],
                dst_ref=recv_hbm.at[pl.ds(c * CHUNK, CHUNK), :],
                send_sem=send_x_sems.at[c],
                recv_sem=recv_x_sems.at[c],
                device_id=x_nbr,
                device_id_type=pl.DeviceIdType.MESH,
            )
            rd.start()
            x_rdmas.append(rd)

        def make_y_send(c, s, dst_row0):
            return pltpu.make_async_remote_copy(
                src_ref=o_vmem.at[s],
                dst_ref=out_ref.at[pl.ds(dst_row0 + c * CHUNK, CHUNK), :],
                send_sem=send_y_sems.at[c],
                recv_sem=recv_y_sems.at[c],
                device_id=y_nbr,
                device_id_type=pl.DeviceIdType.MESH,
            )

        def start_inputs(c):
            s = c % 2
            cps = (
                pltpu.make_async_copy(
                    partial_ref.at[0, pl.ds(row0 + c * CHUNK, CHUNK), :],
                    p_vmem.at[s], in_sems.at[s, 0]),
                pltpu.make_async_copy(
                    recv_hbm.at[pl.ds(c * CHUNK, CHUNK), :],
                    q_vmem.at[s], in_sems.at[s, 1]),
                pltpu.make_async_copy(
                    resid_ref.at[pl.ds(row0 + c * CHUNK, CHUNK), :],
                    r_vmem.at[s], in_sems.at[s, 2]),
            )
            for cp in cps:
                cp.start()
            return cps

        x_rdmas[0].wait_recv()
        pending_in = {0: start_inputs(0)}
        y_sends = {}
        out_cps = {}
        for c in range(N_CHUNKS):
            if c + 1 < N_CHUNKS:
                x_rdmas[c + 1].wait_recv()
                pending_in[c + 1] = start_inputs(c + 1)
            for cp in pending_in.pop(c):
                cp.wait()
            s = c % 2
            if c >= 2:
                y_sends[c - 2].wait_send()
                out_cps[c - 2].wait()
            y = p_vmem[s] + q_vmem[s] + r_vmem[s]
            rms = jnp.sqrt(jnp.mean(y * y, axis=-1, keepdims=True) + 1e-6)
            o_vmem[s] = y / rms * gamma_ref[...]
            cp_o = pltpu.make_async_copy(
                o_vmem.at[s],
                out_ref.at[pl.ds(row0 + c * CHUNK, CHUNK), :],
                out_sems.at[s])
            cp_o.start()
            out_cps[c] = cp_o
            rd = make_y_send(c, s, row0)
            rd.start()
            y_sends[c] = rd

        for c in (N_CHUNKS - 2, N_CHUNKS - 1):
            y_sends[c].wait_send()
            out_cps[c].wait()
        for c in range(N_CHUNKS):
            x_rdmas[c].wait_send()
        for c in range(N_CHUNKS):
            make_y_send(c, 0, other_row0).wait_recv()

    gamma2 = gamma.reshape(1, D)
    out, _ = pl.pallas_call(
        body,
        out_shape=[
            jax.ShapeDtypeStruct((M, D), jnp.float32),
            jax.ShapeDtypeStruct((HALF, D), jnp.float32),
        ],
        in_specs=[
            pl.BlockSpec(memory_space=pltpu.HBM),
            pl.BlockSpec(memory_space=pltpu.HBM),
            pl.BlockSpec(memory_space=pltpu.VMEM),
        ],
        out_specs=[
            pl.BlockSpec(memory_space=pltpu.HBM),
            pl.BlockSpec(memory_space=pltpu.HBM),
        ],
        scratch_shapes=[
            pltpu.VMEM((2, CHUNK, D), jnp.float32),
            pltpu.VMEM((2, CHUNK, D), jnp.float32),
            pltpu.VMEM((2, CHUNK, D), jnp.float32),
            pltpu.VMEM((2, CHUNK, D), jnp.float32),
            pltpu.SemaphoreType.DMA((2, 3)),
            pltpu.SemaphoreType.DMA((2,)),
            pltpu.SemaphoreType.DMA((N_CHUNKS,)),
            pltpu.SemaphoreType.DMA((N_CHUNKS,)),
            pltpu.SemaphoreType.DMA((N_CHUNKS,)),
            pltpu.SemaphoreType.DMA((N_CHUNKS,)),
        ],
        compiler_params=pltpu.CompilerParams(collective_id=0),
    )(partial, resid, gamma2)
    return out


# device time: 434909 ns/iter; 1.1686x vs baseline; 1.0506x over previous
import jax
import jax.numpy as jnp
from jax import lax
from jax.experimental import pallas as pl
from jax.experimental.pallas import tpu as pltpu

M = 4096
D = 4096
HALF = M // 2
N_CHUNKS = 16
CHUNK = HALF // N_CHUNKS


def kernel(partial, resid, gamma):
    def body(partial_ref, resid_ref, gamma_ref, out_ref,
             p_vmem, q_vmem, r_vmem, o_vmem,
             in_sems, out_sems,
             send_x_sems, recv_x_sems, send_y_sems, recv_y_sems):
        my_x = lax.axis_index("x")
        my_y = lax.axis_index("y")
        x_nbr = (1 - my_x, my_y)
        y_nbr = (my_x, 1 - my_y)

        barrier_sem = pltpu.get_barrier_semaphore()
        for nbr in (x_nbr, y_nbr):
            pl.semaphore_signal(
                barrier_sem, inc=1,
                device_id=nbr, device_id_type=pl.DeviceIdType.MESH,
            )
        pl.semaphore_wait(barrier_sem, 2)

        row0 = my_y * HALF
        other_row0 = (1 - my_y) * HALF

        x_rdmas = []
        for c in range(N_CHUNKS):
            rd = pltpu.make_async_remote_copy(
                src_ref=partial_ref.at[0, pl.ds(row0 + c * CHUNK, CHUNK), :],
                dst_ref=q_vmem.at[c],
                send_sem=send_x_sems.at[c],
                recv_sem=recv_x_sems.at[c],
                device_id=x_nbr,
                device_id_type=pl.DeviceIdType.MESH,
            )
            rd.start()
            x_rdmas.append(rd)

        def make_y_send(c, s, dst_row0):
            return pltpu.make_async_remote_copy(
                src_ref=o_vmem.at[s],
                dst_ref=out_ref.at[pl.ds(dst_row0 + c * CHUNK, CHUNK), :],
                send_sem=send_y_sems.at[c],
                recv_sem=recv_y_sems.at[c],
                device_id=y_nbr,
                device_id_type=pl.DeviceIdType.MESH,
            )

        def start_inputs(c):
            s = c % 2
            cps = (
                pltpu.make_async_copy(
                    partial_ref.at[0, pl.ds(row0 + c * CHUNK, CHUNK), :],
                    p_vmem.at[s], in_sems.at[s, 0]),
                pltpu.make_async_copy(
                    resid_ref.at[pl.ds(row0 + c * CHUNK, CHUNK), :],
                    r_vmem.at[s], in_sems.at[s, 1]),
            )
            for cp in cps:
                cp.start()
            return cps

        pending_in = {0: start_inputs(0), 1: start_inputs(1)}
        y_sends = {}
        out_cps = {}
        for c in range(N_CHUNKS):
            for cp in pending_in.pop(c):
                cp.wait()
            x_rdmas[c].wait_recv()
            s = c % 2
            if c >= 2:
                y_sends[c - 2].wait_send()
                out_cps[c - 2].wait()
            y = p_vmem[s] + q_vmem[c] + r_vmem[s]
            rms = jnp.sqrt(jnp.mean(y * y, axis=-1, keepdims=True) + 1e-6)
            o_vmem[s] = y / rms * gamma_ref[...]
            if c + 2 < N_CHUNKS:
                pending_in[c + 2] = start_inputs(c + 2)
            cp_o = pltpu.make_async_copy(
                o_vmem.at[s],
                out_ref.at[pl.ds(row0 + c * CHUNK, CHUNK), :],
                out_sems.at[s])
            cp_o.start()
            out_cps[c] = cp_o
            rd = make_y_send(c, s, row0)
            rd.start()
            y_sends[c] = rd

        for c in (N_CHUNKS - 2, N_CHUNKS - 1):
            y_sends[c].wait_send()
            out_cps[c].wait()
        for c in range(N_CHUNKS):
            x_rdmas[c].wait_send()
        for c in range(N_CHUNKS):
            make_y_send(c, 0, other_row0).wait_recv()

    gamma2 = gamma.reshape(1, D)
    return pl.pallas_call(
        body,
        out_shape=jax.ShapeDtypeStruct((M, D), jnp.float32),
        in_specs=[
            pl.BlockSpec(memory_space=pltpu.HBM),
            pl.BlockSpec(memory_space=pltpu.HBM),
            pl.BlockSpec(memory_space=pltpu.VMEM),
        ],
        out_specs=pl.BlockSpec(memory_space=pltpu.HBM),
        scratch_shapes=[
            pltpu.VMEM((2, CHUNK, D), jnp.float32),
            pltpu.VMEM((N_CHUNKS, CHUNK, D), jnp.float32),
            pltpu.VMEM((2, CHUNK, D), jnp.float32),
            pltpu.VMEM((2, CHUNK, D), jnp.float32),
            pltpu.SemaphoreType.DMA((2, 2)),
            pltpu.SemaphoreType.DMA((2,)),
            pltpu.SemaphoreType.DMA((N_CHUNKS,)),
            pltpu.SemaphoreType.DMA((N_CHUNKS,)),
            pltpu.SemaphoreType.DMA((N_CHUNKS,)),
            pltpu.SemaphoreType.DMA((N_CHUNKS,)),
        ],
        compiler_params=pltpu.CompilerParams(
            collective_id=0,
            vmem_limit_bytes=60 * 1024 * 1024,
        ),
    )(partial, resid, gamma2)


# device time: 423922 ns/iter; 1.1989x vs baseline; 1.0259x over previous
import jax
import jax.numpy as jnp
from jax import lax
from jax.experimental import pallas as pl
from jax.experimental.pallas import tpu as pltpu

M = 4096
D = 4096
HALF = M // 2
N_CHUNKS = 32
CHUNK = HALF // N_CHUNKS


def kernel(partial, resid, gamma):
    def body(partial_ref, resid_ref, gamma_ref, out_ref,
             p_vmem, q_vmem, r_vmem, o_vmem,
             in_sems, out_sems,
             send_x_sems, recv_x_sems, send_y_sems, recv_y_sems):
        my_x = lax.axis_index("x")
        my_y = lax.axis_index("y")
        x_nbr = (1 - my_x, my_y)
        y_nbr = (my_x, 1 - my_y)

        barrier_sem = pltpu.get_barrier_semaphore()
        for nbr in (x_nbr, y_nbr):
            pl.semaphore_signal(
                barrier_sem, inc=1,
                device_id=nbr, device_id_type=pl.DeviceIdType.MESH,
            )
        pl.semaphore_wait(barrier_sem, 2)

        row0 = my_y * HALF
        other_row0 = (1 - my_y) * HALF

        x_rdmas = []
        for c in range(N_CHUNKS):
            rd = pltpu.make_async_remote_copy(
                src_ref=partial_ref.at[0, pl.ds(row0 + c * CHUNK, CHUNK), :],
                dst_ref=q_vmem.at[c],
                send_sem=send_x_sems.at[c],
                recv_sem=recv_x_sems.at[c],
                device_id=x_nbr,
                device_id_type=pl.DeviceIdType.MESH,
            )
            rd.start()
            x_rdmas.append(rd)

        def make_y_send(c, s, dst_row0):
            return pltpu.make_async_remote_copy(
                src_ref=o_vmem.at[s],
                dst_ref=out_ref.at[pl.ds(dst_row0 + c * CHUNK, CHUNK), :],
                send_sem=send_y_sems.at[c],
                recv_sem=recv_y_sems.at[c],
                device_id=y_nbr,
                device_id_type=pl.DeviceIdType.MESH,
            )

        def start_inputs(c):
            s = c % 2
            cps = (
                pltpu.make_async_copy(
                    partial_ref.at[0, pl.ds(row0 + c * CHUNK, CHUNK), :],
                    p_vmem.at[s], in_sems.at[s, 0]),
                pltpu.make_async_copy(
                    resid_ref.at[pl.ds(row0 + c * CHUNK, CHUNK), :],
                    r_vmem.at[s], in_sems.at[s, 1]),
            )
            for cp in cps:
                cp.start()
            return cps

        pending_in = {0: start_inputs(0), 1: start_inputs(1)}
        y_sends = {}
        out_cps = {}
        for c in range(N_CHUNKS):
            for cp in pending_in.pop(c):
                cp.wait()
            x_rdmas[c].wait_recv()
            s = c % 2
            if c >= 2:
                y_sends[c - 2].wait_send()
                out_cps[c - 2].wait()
            y = p_vmem[s] + q_vmem[c] + r_vmem[s]
            inv = lax.rsqrt(jnp.mean(y * y, axis=-1, keepdims=True) + 1e-6)
            o_vmem[s] = y * inv * gamma_ref[...]
            if c + 2 < N_CHUNKS:
                pending_in[c + 2] = start_inputs(c + 2)
            cp_o = pltpu.make_async_copy(
                o_vmem.at[s],
                out_ref.at[pl.ds(row0 + c * CHUNK, CHUNK), :],
                out_sems.at[s])
            cp_o.start()
            out_cps[c] = cp_o
            rd = make_y_send(c, s, row0)
            rd.start()
            y_sends[c] = rd

        for c in (N_CHUNKS - 2, N_CHUNKS - 1):
            y_sends[c].wait_send()
            out_cps[c].wait()
        for c in range(N_CHUNKS):
            x_rdmas[c].wait_send()
        for c in range(N_CHUNKS):
            make_y_send(c, 0, other_row0).wait_recv()

    gamma2 = gamma.reshape(1, D)
    return pl.pallas_call(
        body,
        out_shape=jax.ShapeDtypeStruct((M, D), jnp.float32),
        in_specs=[
            pl.BlockSpec(memory_space=pltpu.HBM),
            pl.BlockSpec(memory_space=pltpu.HBM),
            pl.BlockSpec(memory_space=pltpu.VMEM),
        ],
        out_specs=pl.BlockSpec(memory_space=pltpu.HBM),
        scratch_shapes=[
            pltpu.VMEM((2, CHUNK, D), jnp.float32),
            pltpu.VMEM((N_CHUNKS, CHUNK, D), jnp.float32),
            pltpu.VMEM((2, CHUNK, D), jnp.float32),
            pltpu.VMEM((2, CHUNK, D), jnp.float32),
            pltpu.SemaphoreType.DMA((2, 2)),
            pltpu.SemaphoreType.DMA((2,)),
            pltpu.SemaphoreType.DMA((N_CHUNKS,)),
            pltpu.SemaphoreType.DMA((N_CHUNKS,)),
            pltpu.SemaphoreType.DMA((N_CHUNKS,)),
            pltpu.SemaphoreType.DMA((N_CHUNKS,)),
        ],
        compiler_params=pltpu.CompilerParams(
            collective_id=0,
            vmem_limit_bytes=60 * 1024 * 1024,
        ),
    )(partial, resid, gamma2)


# device time: 419072 ns/iter; 1.2128x vs baseline; 1.0116x over previous
import jax
import jax.numpy as jnp
from jax import lax
from jax.experimental import pallas as pl
from jax.experimental.pallas import tpu as pltpu

M = 4096
D = 4096
HALF = M // 2
N_CHUNKS = 64
CHUNK = HALF // N_CHUNKS


def kernel(partial, resid, gamma):
    def body(partial_ref, resid_ref, gamma_ref, out_ref,
             p_vmem, q_vmem, r_vmem, o_vmem,
             in_sems, out_sems,
             send_x_sems, recv_x_sems, send_y_sems, recv_y_sems):
        my_x = lax.axis_index("x")
        my_y = lax.axis_index("y")
        x_nbr = (1 - my_x, my_y)
        y_nbr = (my_x, 1 - my_y)

        barrier_sem = pltpu.get_barrier_semaphore()
        for nbr in (x_nbr, y_nbr):
            pl.semaphore_signal(
                barrier_sem, inc=1,
                device_id=nbr, device_id_type=pl.DeviceIdType.MESH,
            )
        pl.semaphore_wait(barrier_sem, 2)

        row0 = my_y * HALF
        other_row0 = (1 - my_y) * HALF

        x_rdmas = []
        for c in range(N_CHUNKS):
            rd = pltpu.make_async_remote_copy(
                src_ref=partial_ref.at[0, pl.ds(row0 + c * CHUNK, CHUNK), :],
                dst_ref=q_vmem.at[c],
                send_sem=send_x_sems.at[c],
                recv_sem=recv_x_sems.at[c],
                device_id=x_nbr,
                device_id_type=pl.DeviceIdType.MESH,
            )
            rd.start()
            x_rdmas.append(rd)

        def make_y_send(c, s, dst_row0):
            return pltpu.make_async_remote_copy(
                src_ref=o_vmem.at[s],
                dst_ref=out_ref.at[pl.ds(dst_row0 + c * CHUNK, CHUNK), :],
                send_sem=send_y_sems.at[c],
                recv_sem=recv_y_sems.at[c],
                device_id=y_nbr,
                device_id_type=pl.DeviceIdType.MESH,
            )

        def start_inputs(c):
            s = c % 2
            cps = (
                pltpu.make_async_copy(
                    partial_ref.at[0, pl.ds(row0 + c * CHUNK, CHUNK), :],
                    p_vmem.at[s], in_sems.at[s, 0]),
                pltpu.make_async_copy(
                    resid_ref.at[pl.ds(row0 + c * CHUNK, CHUNK), :],
                    r_vmem.at[s], in_sems.at[s, 1]),
            )
            for cp in cps:
                cp.start()
            return cps

        pending_in = {0: start_inputs(0), 1: start_inputs(1)}
        y_sends = {}
        out_cps = {}
        for c in range(N_CHUNKS):
            for cp in pending_in.pop(c):
                cp.wait()
            x_rdmas[c].wait_recv()
            s = c % 2
            if c >= 2:
                y_sends[c - 2].wait_send()
                out_cps[c - 2].wait()
            y = p_vmem[s] + q_vmem[c] + r_vmem[s]
            inv = lax.rsqrt(jnp.mean(y * y, axis=-1, keepdims=True) + 1e-6)
            o_vmem[s] = y * inv * gamma_ref[...]
            if c + 2 < N_CHUNKS:
                pending_in[c + 2] = start_inputs(c + 2)
            cp_o = pltpu.make_async_copy(
                o_vmem.at[s],
                out_ref.at[pl.ds(row0 + c * CHUNK, CHUNK), :],
                out_sems.at[s])
            cp_o.start()
            out_cps[c] = cp_o
            rd = make_y_send(c, s, row0)
            rd.start()
            y_sends[c] = rd

        for c in (N_CHUNKS - 2, N_CHUNKS - 1):
            y_sends[c].wait_send()
            out_cps[c].wait()
        for c in range(N_CHUNKS):
            x_rdmas[c].wait_send()
        for c in range(N_CHUNKS):
            make_y_send(c, 0, other_row0).wait_recv()

    gamma2 = gamma.reshape(1, D)
    return pl.pallas_call(
        body,
        out_shape=jax.ShapeDtypeStruct((M, D), jnp.float32),
        in_specs=[
            pl.BlockSpec(memory_space=pltpu.HBM),
            pl.BlockSpec(memory_space=pltpu.HBM),
            pl.BlockSpec(memory_space=pltpu.VMEM),
        ],
        out_specs=pl.BlockSpec(memory_space=pltpu.HBM),
        scratch_shapes=[
            pltpu.VMEM((2, CHUNK, D), jnp.float32),
            pltpu.VMEM((N_CHUNKS, CHUNK, D), jnp.float32),
            pltpu.VMEM((2, CHUNK, D), jnp.float32),
            pltpu.VMEM((2, CHUNK, D), jnp.float32),
            pltpu.SemaphoreType.DMA((2, 2)),
            pltpu.SemaphoreType.DMA((2,)),
            pltpu.SemaphoreType.DMA((N_CHUNKS,)),
            pltpu.SemaphoreType.DMA((N_CHUNKS,)),
            pltpu.SemaphoreType.DMA((N_CHUNKS,)),
            pltpu.SemaphoreType.DMA((N_CHUNKS,)),
        ],
        compiler_params=pltpu.CompilerParams(
            collective_id=0,
            vmem_limit_bytes=60 * 1024 * 1024,
        ),
    )(partial, resid, gamma2)
